# baseline (speedup 1.0000x reference)
import functools

import jax
import jax.numpy as jnp
from jax import lax
from jax.experimental import pallas as pl
from jax.experimental.pallas import tpu as pltpu
from jax.experimental.pallas import tpu_sc as plsc

BATCH = 16384
HIDDEN = 64
NB = 6


@functools.cache
def _make_kernels(B, D, V):
    info = plsc.get_sparse_core_info()
    NC, NS, L = info.num_cores, info.num_subcores, info.num_lanes
    NW = NC * NS
    b_per_w = B // NW
    blocks = (V + 127) // 128
    bpt = -(-blocks // NW)
    rounds = -(-bpt // NB)
    mesh = plsc.VectorSubcoreMesh(core_axis_name="c", subcore_axis_name="s")
    params = pltpu.CompilerParams(
        use_tc_tiling_on_sc=True, needs_layout_passes=False)

    @functools.partial(
        pl.kernel,
        mesh=mesh,
        out_type=jax.ShapeDtypeStruct((B + 16, 128), jnp.float32),
        compiler_params=params,
        scratch_types=[
            pltpu.VMEM((B,), jnp.int32),
            pltpu.VMEM((B + 16,), jnp.int32),
            pltpu.VMEM((B + 16,), jnp.int32),
            pltpu.VMEM((NB, D, 128), jnp.float32),
            pltpu.VMEM((32,), jnp.int32),
            pltpu.VMEM((32,), jnp.int32),
            pltpu.VMEM((16, 128), jnp.float32),
            pltpu.VMEM((16,), jnp.int32),
            pltpu.SMEM((1,), jnp.int32),
            pltpu.SemaphoreType.DMA,
        ],
    )
    def p1(labels_hbm, tT_hbm, rows_hbm, alllab_v, wlab_v, wpos_v, blocks_v,
           mlab_v, mpos_v, rowbuf_v, idxbuf_v, nrow_s, sem):
        wid = lax.axis_index("s") * NC + lax.axis_index("c")
        lo = wid * bpt
        nblk = jnp.minimum(lo + bpt, blocks) - lo
        pltpu.sync_copy(labels_hbm, alllab_v)
        lane = lax.iota(jnp.int32, L)
        dump = jnp.full((L,), B, jnp.int32)
        idxbuf_v[...] = dump
        nrow_s[0] = 0

        def scan_body(p, cur):
            v = alllab_v[pl.ds(p * L, L)]
            m = ((v >> 7) >= lo) & ((v >> 7) < lo + nblk)
            c = plsc.cumsum(m.astype(jnp.int32))
            at = jnp.full((L,), cur, jnp.int32) + c - 1
            plsc.store_scatter(wlab_v, [at], v, mask=m)
            plsc.store_scatter(wpos_v, [at], lane + p * L, mask=m)
            return cur + c[15]

        n = lax.fori_loop(0, B // L, scan_body, 0)
        nv = (n + L - 1) // L

        def bdesc(kb, u):
            off = pl.multiple_of((lo + kb) * 128, 128)
            return pltpu.make_async_copy(
                tT_hbm.at[:, pl.ds(off, 128)], blocks_v.at[u], sem)

        def flush():
            pltpu.sync_copy(rowbuf_v, rows_hbm.at[idxbuf_v])
            idxbuf_v[...] = dump

        def process(u):
            def eat(e, _):
                lab = mlab_v[pl.ds(e, L)][0]
                b = mpos_v[pl.ds(e, L)][0]
                lsp = jnp.full((L,), lab & 127, jnp.int32)
                nr = nrow_s[0]
                nrsp = jnp.full((L,), nr, jnp.int32)
                for cg in range(D // L):
                    cvec = lane + cg * L
                    val = plsc.load_gather(blocks_v.at[u], [cvec, lsp])
                    plsc.store_scatter(rowbuf_v, [nrsp, cvec], val)
                plsc.store_scatter(idxbuf_v, [nrsp],
                                   jnp.full((L,), b, jnp.int32),
                                   mask=lane == 0)
                nrow_s[0] = nr + 1

                @pl.when(nr + 1 == 16)
                def _():
                    flush()
                    nrow_s[0] = 0
                return 0
            return eat

        def match_block(u, k_id):
            def mvec(q, _):
                wv = wlab_v[pl.ds(q * L, L)]
                wp = wpos_v[pl.ds(q * L, L)]
                valid = (lane + q * L) < n
                m = ((wv >> 7) == k_id) & valid
                c = plsc.cumsum(m.astype(jnp.int32))
                cnt = c[15]

                @pl.when(cnt > 0)
                def _():
                    plsc.store_scatter(mlab_v, [c - 1], wv, mask=m)
                    plsc.store_scatter(mpos_v, [c - 1], wp, mask=m)
                    lax.fori_loop(0, cnt, process(u), 0)
                return 0
            lax.fori_loop(0, nv, mvec, 0)

        for u in range(NB):
            bdesc(u, u).start()

        def round_body(r, _):
            for u in range(NB):
                kb = r * NB + u

                @pl.when(kb < nblk)
                def _(kb=kb, u=u):
                    bdesc(kb, u).wait()
                    match_block(u, lo + kb)

                    @pl.when(kb + NB < nblk)
                    def _():
                        bdesc(kb + NB, u).start()
            return 0

        lax.fori_loop(0, rounds, round_body, 0)
        flush()

    @functools.partial(
        pl.kernel,
        mesh=mesh,
        out_type=jax.ShapeDtypeStruct((D, B), jnp.float32),
        compiler_params=params,
        scratch_types=[
            pltpu.VMEM((b_per_w, 128), jnp.float32),
            pltpu.VMEM((b_per_w // 128, D, 128), jnp.float32),
            pltpu.SemaphoreType.DMA,
        ],
    )
    def p2(rows_hbm, outT_hbm, rbuf_v, obuf_v, sem):
        wid = lax.axis_index("s") * NC + lax.axis_index("c")
        base = wid * b_per_w
        pltpu.sync_copy(rows_hbm.at[pl.ds(base, b_per_w)], rbuf_v)
        lane = lax.iota(jnp.int32, L)
        for g in range(b_per_w // 128):
            gsp = jnp.full((L,), g, jnp.int32)

            def cbody(cc, _, gsp=gsp, g=g):
                csp = jnp.full((L,), cc, jnp.int32)
                for jg in range(128 // L):
                    jvec = lane + (g * 128 + jg * L)
                    val = plsc.load_gather(rbuf_v, [jvec, csp])
                    plsc.store_scatter(
                        obuf_v, [gsp, csp, lane + jg * L], val)
                return 0

            lax.fori_loop(0, D, cbody, 0)
        outs = [
            pltpu.async_copy(obuf_v.at[g],
                             outT_hbm.at[:, pl.ds(base + g * 128, 128)], sem)
            for g in range(b_per_w // 128)
        ]
        for o in outs:
            o.wait()

    return p1, p2


def kernel(labels, train, table):
    p1, p2 = _make_kernels(BATCH, HIDDEN, table.shape[0])
    rows = p1(labels.astype(jnp.int32), table.T)
    outT = p2(rows)
    return outT.T


# speedup vs baseline: 1.7809x; 1.7809x over previous
import functools

import jax
import jax.numpy as jnp
from jax import lax
from jax.experimental import pallas as pl
from jax.experimental.pallas import tpu as pltpu
from jax.experimental.pallas import tpu_sc as plsc

BATCH = 16384
HIDDEN = 64
NB = 6


@functools.cache
def _make_kernels(B, D, V):
    info = plsc.get_sparse_core_info()
    NC, NS, L = info.num_cores, info.num_subcores, info.num_lanes
    NW = NC * NS
    b_per_w = B // NW
    blocks = (V + 127) // 128
    bpt = -(-blocks // NW)
    rounds = -(-bpt // NB)
    mesh = plsc.VectorSubcoreMesh(core_axis_name="c", subcore_axis_name="s")
    params = pltpu.CompilerParams(
        use_tc_tiling_on_sc=True, needs_layout_passes=False)

    @functools.partial(
        pl.kernel,
        mesh=mesh,
        out_type=jax.ShapeDtypeStruct((B + 16, 128), jnp.float32),
        compiler_params=params,
        scratch_types=[
            pltpu.VMEM((B + 16,), jnp.int32),
            pltpu.VMEM((B + 16,), jnp.int32),
            pltpu.VMEM((B + 16,), jnp.int32),
            pltpu.VMEM((B + 16,), jnp.int32),
            pltpu.VMEM((NB, D, 128), jnp.float32),
            pltpu.VMEM((272,), jnp.int32),
            pltpu.VMEM((272,), jnp.int32),
            pltpu.VMEM((272,), jnp.int32),
            pltpu.VMEM((272,), jnp.int32),
            pltpu.VMEM((16, 128), jnp.float32),
            pltpu.VMEM((16,), jnp.int32),
            pltpu.SMEM((1,), jnp.int32),
            pltpu.SemaphoreType.DMA,
        ],
    )
    def p1(labels_hbm, tT_hbm, rows_hbm, slab_v, wlab_v, wpos_v, spos_v,
           blocks_v, offs_v, cur_v, dblk_v, dbase_v, rowbuf_v, idxbuf_v,
           nrow_s, sem):
        wid = lax.axis_index("s") * NC + lax.axis_index("c")
        lo = wid * bpt
        nblk = jnp.minimum(lo + bpt, blocks) - lo
        pltpu.sync_copy(labels_hbm, slab_v.at[pl.ds(0, B)])
        lane = lax.iota(jnp.int32, L)
        dump = jnp.full((L,), B, jnp.int32)
        idxbuf_v[...] = dump
        nrow_s[0] = 0
        zeros = jnp.zeros((L,), jnp.int32)
        for kz in range(272 // L):
            offs_v[pl.ds(kz * L, L)] = zeros
            cur_v[pl.ds(kz * L, L)] = zeros

        def at1(ref, j):
            return ref[pl.ds(j, L)][0]

        def put1(ref, idx, val):
            plsc.store_scatter(ref, [jnp.full((L,), idx, jnp.int32)],
                               jnp.full((L,), val, jnp.int32),
                               mask=lane == 0)

        def scan_body(p, cur):
            v = slab_v[pl.ds(p * L, L)]
            m = ((v >> 7) >= lo) & ((v >> 7) < lo + nblk)
            c = plsc.cumsum(m.astype(jnp.int32))
            at = jnp.full((L,), cur, jnp.int32) + c - 1
            plsc.store_scatter(wlab_v, [at], v, mask=m)
            plsc.store_scatter(wpos_v, [at], lane + p * L, mask=m)
            return cur + c[15]

        n = lax.fori_loop(0, B // L, scan_body, 0)

        def count_body(j, _):
            blk = (at1(wlab_v, j) >> 7) - lo
            put1(cur_v, blk, at1(cur_v, blk) + 1)
            return 0

        lax.fori_loop(0, n, count_body, 0)

        def psum_body(kz, carry):
            tot, nd = carry
            v = cur_v[pl.ds(kz * L, L)]
            c = plsc.cumsum(v)
            excl = c - v + jnp.full((L,), tot, jnp.int32)
            offs_v[pl.ds(kz * L, L)] = excl
            m = v > 0
            dc = plsc.cumsum(m.astype(jnp.int32))
            dat = jnp.full((L,), nd, jnp.int32) + dc - 1
            plsc.store_scatter(dblk_v, [dat], lane + kz * L, mask=m)
            plsc.store_scatter(dbase_v, [dat], excl, mask=m)
            return tot + c[15], nd + dc[15]

        _, ndist = lax.fori_loop(0, 272 // L, psum_body, (0, 0))
        put1(dbase_v, ndist, n)
        for kz in range(272 // L):
            cur_v[pl.ds(kz * L, L)] = zeros

        def place_body(j, _):
            lab = at1(wlab_v, j)
            blk = (lab >> 7) - lo
            dst = at1(offs_v, blk) + at1(cur_v, blk)
            put1(slab_v, dst, lab)
            put1(spos_v, dst, at1(wpos_v, j))
            put1(cur_v, blk, at1(cur_v, blk) + 1)
            return 0

        lax.fori_loop(0, n, place_body, 0)

        def bdesc(f, u):
            off = pl.multiple_of(
                (at1(dblk_v, f) + lo) * 128, 128)
            return pltpu.make_async_copy(
                tT_hbm.at[:, pl.ds(off, 128)], blocks_v.at[u], sem)

        def flush():
            pltpu.sync_copy(rowbuf_v, rows_hbm.at[idxbuf_v])
            idxbuf_v[...] = dump

        def process(u):
            def eat(j, _):
                lab = at1(slab_v, j)
                b = at1(spos_v, j)
                lsp = jnp.full((L,), lab & 127, jnp.int32)
                nr = nrow_s[0]
                nrsp = jnp.full((L,), nr, jnp.int32)
                for cg in range(D // L):
                    cvec = lane + cg * L
                    val = plsc.load_gather(blocks_v.at[u], [cvec, lsp])
                    plsc.store_scatter(rowbuf_v, [nrsp, cvec], val)
                plsc.store_scatter(idxbuf_v, [nrsp],
                                   jnp.full((L,), b, jnp.int32),
                                   mask=lane == 0)
                nrow_s[0] = nr + 1

                @pl.when(nr + 1 == 16)
                def _():
                    flush()
                    nrow_s[0] = 0
                return 0
            return eat

        for u in range(NB):
            @pl.when(u < ndist)
            def _(u=u):
                bdesc(u, u).start()

        def round_body(r, _):
            for u in range(NB):
                f = r * NB + u

                @pl.when(f < ndist)
                def _(f=f, u=u):
                    bdesc(f, u).wait()
                    lax.fori_loop(at1(dbase_v, f), at1(dbase_v, f + 1),
                                  process(u), 0)

                    @pl.when(f + NB < ndist)
                    def _():
                        bdesc(f + NB, u).start()
            return 0

        lax.fori_loop(0, rounds, round_body, 0)
        flush()

    @functools.partial(
        pl.kernel,
        mesh=mesh,
        out_type=jax.ShapeDtypeStruct((D, B), jnp.float32),
        compiler_params=params,
        scratch_types=[
            pltpu.VMEM((b_per_w, 128), jnp.float32),
            pltpu.VMEM((b_per_w // 128, D, 128), jnp.float32),
            pltpu.SemaphoreType.DMA,
        ],
    )
    def p2(rows_hbm, outT_hbm, rbuf_v, obuf_v, sem):
        wid = lax.axis_index("s") * NC + lax.axis_index("c")
        base = wid * b_per_w
        pltpu.sync_copy(rows_hbm.at[pl.ds(base, b_per_w)], rbuf_v)
        lane = lax.iota(jnp.int32, L)
        for g in range(b_per_w // 128):
            gsp = jnp.full((L,), g, jnp.int32)

            def cbody(cc, _, gsp=gsp, g=g):
                csp = jnp.full((L,), cc, jnp.int32)
                for jg in range(128 // L):
                    jvec = lane + (g * 128 + jg * L)
                    val = plsc.load_gather(rbuf_v, [jvec, csp])
                    plsc.store_scatter(
                        obuf_v, [gsp, csp, lane + jg * L], val)
                return 0

            lax.fori_loop(0, D, cbody, 0)
        outs = [
            pltpu.async_copy(obuf_v.at[g],
                             outT_hbm.at[:, pl.ds(base + g * 128, 128)], sem)
            for g in range(b_per_w // 128)
        ]
        for o in outs:
            o.wait()

    return p1, p2


def kernel(labels, train, table):
    p1, p2 = _make_kernels(BATCH, HIDDEN, table.shape[0])
    rows = p1(labels.astype(jnp.int32), table.T)
    outT = p2(rows)
    return outT.T


# speedup vs baseline: 1.8240x; 1.0242x over previous
import functools

import jax
import jax.numpy as jnp
from jax import lax
from jax.experimental import pallas as pl
from jax.experimental.pallas import tpu as pltpu
from jax.experimental.pallas import tpu_sc as plsc

BATCH = 16384
HIDDEN = 64
NB = 6


@functools.cache
def _make_kernels(B, D, V):
    info = plsc.get_sparse_core_info()
    NC, NS, L = info.num_cores, info.num_subcores, info.num_lanes
    NW = NC * NS
    b_per_w = B // NW
    blocks = (V + 127) // 128
    bpt = -(-blocks // NW)
    rounds = -(-bpt // NB)
    mesh = plsc.VectorSubcoreMesh(core_axis_name="c", subcore_axis_name="s")
    params = pltpu.CompilerParams(
        use_tc_tiling_on_sc=True, needs_layout_passes=False)

    @functools.partial(
        pl.kernel,
        mesh=mesh,
        out_type=jax.ShapeDtypeStruct((B + 16, 128), jnp.float32),
        compiler_params=params,
        scratch_types=[
            pltpu.VMEM((B + 16,), jnp.int32),
            pltpu.VMEM((B + 16,), jnp.int32),
            pltpu.VMEM((B + 16,), jnp.int32),
            pltpu.VMEM((B + 16,), jnp.int32),
            pltpu.VMEM((NB, D, 128), jnp.float32),
            pltpu.VMEM((272,), jnp.int32),
            pltpu.VMEM((272,), jnp.int32),
            pltpu.VMEM((272,), jnp.int32),
            pltpu.VMEM((272,), jnp.int32),
            pltpu.VMEM((2, 16, 128), jnp.float32),
            pltpu.VMEM((2, 16), jnp.int32),
            pltpu.SMEM((2,), jnp.int32),
            pltpu.SemaphoreType.DMA,
            pltpu.SemaphoreType.DMA,
            pltpu.SemaphoreType.DMA,
        ],
    )
    def p1(labels_hbm, tT_hbm, rows_hbm, slab_v, wlab_v, wpos_v, spos_v,
           blocks_v, offs_v, cur_v, dblk_v, dbase_v, rowbuf_v, idxbuf_v,
           nrow_s, sem, fsem0, fsem1):
        wid = lax.axis_index("s") * NC + lax.axis_index("c")
        lo = wid * bpt
        nblk = jnp.minimum(lo + bpt, blocks) - lo
        pltpu.sync_copy(labels_hbm, slab_v.at[pl.ds(0, B)])
        lane = lax.iota(jnp.int32, L)
        dump = jnp.full((L,), B, jnp.int32)
        idxbuf_v[0, :] = dump
        idxbuf_v[1, :] = dump
        nrow_s[0] = 0
        nrow_s[1] = 0
        zeros = jnp.zeros((L,), jnp.int32)
        for kz in range(272 // L):
            offs_v[pl.ds(kz * L, L)] = zeros
            cur_v[pl.ds(kz * L, L)] = zeros

        def at1(ref, j):
            return ref[pl.ds(j, L)][0]

        def put1(ref, idx, val):
            plsc.store_scatter(ref, [jnp.full((L,), idx, jnp.int32)],
                               jnp.full((L,), val, jnp.int32),
                               mask=lane == 0)

        def scan_body(p, cur):
            v = slab_v[pl.ds(p * L, L)]
            m = ((v >> 7) >= lo) & ((v >> 7) < lo + nblk)
            c = plsc.cumsum(m.astype(jnp.int32))
            at = jnp.full((L,), cur, jnp.int32) + c - 1
            plsc.store_scatter(wlab_v, [at], v, mask=m)
            plsc.store_scatter(wpos_v, [at], lane + p * L, mask=m)
            return cur + c[15]

        n = lax.fori_loop(0, B // L, scan_body, 0)

        def count_body(j, _):
            blk = (at1(wlab_v, j) >> 7) - lo
            put1(cur_v, blk, at1(cur_v, blk) + 1)
            return 0

        lax.fori_loop(0, n, count_body, 0)

        def psum_body(kz, carry):
            tot, nd = carry
            v = cur_v[pl.ds(kz * L, L)]
            c = plsc.cumsum(v)
            excl = c - v + jnp.full((L,), tot, jnp.int32)
            offs_v[pl.ds(kz * L, L)] = excl
            m = v > 0
            dc = plsc.cumsum(m.astype(jnp.int32))
            dat = jnp.full((L,), nd, jnp.int32) + dc - 1
            plsc.store_scatter(dblk_v, [dat], lane + kz * L, mask=m)
            plsc.store_scatter(dbase_v, [dat], excl, mask=m)
            return tot + c[15], nd + dc[15]

        _, ndist = lax.fori_loop(0, 272 // L, psum_body, (0, 0))
        put1(dbase_v, ndist, n)
        for kz in range(272 // L):
            cur_v[pl.ds(kz * L, L)] = zeros

        def place_body(j, _):
            lab = at1(wlab_v, j)
            blk = (lab >> 7) - lo
            dst = at1(offs_v, blk) + at1(cur_v, blk)
            put1(slab_v, dst, lab)
            put1(spos_v, dst, at1(wpos_v, j))
            put1(cur_v, blk, at1(cur_v, blk) + 1)
            return 0

        lax.fori_loop(0, n, place_body, 0)

        def bdesc(f, u):
            off = pl.multiple_of(
                (at1(dblk_v, f) + lo) * 128, 128)
            return pltpu.make_async_copy(
                tT_hbm.at[:, pl.ds(off, 128)], blocks_v.at[u], sem)

        fsems = (fsem0, fsem1)

        def fdesc(pr, fs):
            return pltpu.make_async_copy(
                rowbuf_v.at[pr], rows_hbm.at[idxbuf_v.at[pr]], fs)

        def flush():
            fc = nrow_s[1]
            for pr in range(2):
                @pl.when((fc & 1) == pr)
                def _(pr=pr):
                    @pl.when(fc >= 2)
                    def _():
                        fdesc(pr, fsems[pr]).wait()
                    fdesc(pr, fsems[pr]).start()
            nrow_s[1] = fc + 1

        def fill_pair():
            return nrow_s[1] & 1

        def process(u):
            def eat(j, _):
                lab = at1(slab_v, j)
                b = at1(spos_v, j)
                lsp = jnp.full((L,), lab & 127, jnp.int32)
                nr = nrow_s[0]
                nrsp = jnp.full((L,), nr, jnp.int32)
                prsp = jnp.full((L,), fill_pair(), jnp.int32)
                for cg in range(D // L):
                    cvec = lane + cg * L
                    val = plsc.load_gather(blocks_v.at[u], [cvec, lsp])
                    plsc.store_scatter(rowbuf_v, [prsp, nrsp, cvec], val)
                plsc.store_scatter(idxbuf_v, [prsp, nrsp],
                                   jnp.full((L,), b, jnp.int32),
                                   mask=lane == 0)
                nrow_s[0] = nr + 1

                @pl.when(nr + 1 == 16)
                def _():
                    flush()
                    nrow_s[0] = 0
                return 0
            return eat

        for u in range(NB):
            @pl.when(u < ndist)
            def _(u=u):
                bdesc(u, u).start()

        def round_body(r, _):
            for u in range(NB):
                f = r * NB + u

                @pl.when(f < ndist)
                def _(f=f, u=u):
                    bdesc(f, u).wait()
                    lax.fori_loop(at1(dbase_v, f), at1(dbase_v, f + 1),
                                  process(u), 0)

                    @pl.when(f + NB < ndist)
                    def _():
                        bdesc(f + NB, u).start()
            return 0

        lax.fori_loop(0, rounds, round_body, 0)
        fc = nrow_s[1]

        @pl.when(nrow_s[0] > 0)
        def _():
            pr = fc & 1
            for prs in range(2):
                @pl.when(pr == prs)
                def _(prs=prs):
                    @pl.when(fc >= 2)
                    def _():
                        fdesc(prs, fsems[prs]).wait()
                    plsc.store_scatter(
                        idxbuf_v, [jnp.full((L,), prs, jnp.int32), lane],
                        dump, mask=lane >= nrow_s[0])
                    pltpu.sync_copy(rowbuf_v.at[prs],
                                    rows_hbm.at[idxbuf_v.at[prs]])

        @pl.when(fc >= 1)
        def _():
            for prs in range(2):
                @pl.when(((fc - 1) & 1) == prs)
                def _(prs=prs):
                    fdesc(prs, fsems[prs]).wait()

        @pl.when(fc >= 2)
        def _():
            for prs in range(2):
                @pl.when((fc & 1) == prs)
                def _(prs=prs):
                    @pl.when(nrow_s[0] == 0)
                    def _():
                        fdesc(prs, fsems[prs]).wait()

    @functools.partial(
        pl.kernel,
        mesh=mesh,
        out_type=jax.ShapeDtypeStruct((D, B), jnp.float32),
        compiler_params=params,
        scratch_types=[
            pltpu.VMEM((b_per_w, 128), jnp.float32),
            pltpu.VMEM((b_per_w // 128, D, 128), jnp.float32),
            pltpu.SemaphoreType.DMA,
        ],
    )
    def p2(rows_hbm, outT_hbm, rbuf_v, obuf_v, sem):
        wid = lax.axis_index("s") * NC + lax.axis_index("c")
        base = wid * b_per_w
        loads = [
            pltpu.async_copy(
                rows_hbm.at[pl.ds(base + g * 128, 128)],
                rbuf_v.at[pl.ds(g * 128, 128)], sem)
            for g in range(b_per_w // 128)
        ]
        lane = lax.iota(jnp.int32, L)
        owrites = []
        for g in range(b_per_w // 128):
            loads[g].wait()
            gsp = jnp.full((L,), g, jnp.int32)

            def cbody(cc, _, gsp=gsp, g=g):
                csp = jnp.full((L,), cc, jnp.int32)
                for jg in range(128 // L):
                    jvec = lane + (g * 128 + jg * L)
                    val = plsc.load_gather(rbuf_v, [jvec, csp])
                    plsc.store_scatter(
                        obuf_v, [gsp, csp, lane + jg * L], val)
                return 0

            lax.fori_loop(0, D, cbody, 0)
            owrites.append(pltpu.async_copy(
                obuf_v.at[g],
                outT_hbm.at[:, pl.ds(base + g * 128, 128)], sem))
        for o in owrites:
            o.wait()

    return p1, p2


def kernel(labels, train, table):
    p1, p2 = _make_kernels(BATCH, HIDDEN, table.shape[0])
    rows = p1(labels.astype(jnp.int32), table.T)
    outT = p2(rows)
    return outT.T


# speedup vs baseline: 1.9641x; 1.0768x over previous
import functools

import jax
import jax.numpy as jnp
from jax import lax
from jax.experimental import pallas as pl
from jax.experimental.pallas import tpu as pltpu
from jax.experimental.pallas import tpu_sc as plsc

BATCH = 16384
HIDDEN = 64
NB = 7


@functools.cache
def _make_kernels(B, D, V):
    info = plsc.get_sparse_core_info()
    NC, NS, L = info.num_cores, info.num_subcores, info.num_lanes
    NW = NC * NS
    b_per_w = B // NW
    blocks = (V + 127) // 128
    bpt = -(-blocks // NW)
    rounds = -(-bpt // NB)
    mesh = plsc.VectorSubcoreMesh(core_axis_name="c", subcore_axis_name="s")
    params = pltpu.CompilerParams(
        use_tc_tiling_on_sc=True, needs_layout_passes=False)

    @functools.partial(
        pl.kernel,
        mesh=mesh,
        out_type=jax.ShapeDtypeStruct((B + 16, 128), jnp.float32),
        compiler_params=params,
        scratch_types=[
            pltpu.VMEM((B + 16,), jnp.int32),
            pltpu.VMEM((B + 16,), jnp.int32),
            pltpu.VMEM((B + 16,), jnp.int32),
            pltpu.VMEM((B + 16,), jnp.int32),
            pltpu.VMEM((NB, D, 128), jnp.float32),
            pltpu.VMEM((272,), jnp.int32),
            pltpu.VMEM((272,), jnp.int32),
            pltpu.VMEM((272,), jnp.int32),
            pltpu.VMEM((272,), jnp.int32),
            pltpu.VMEM((2, 16, 128), jnp.float32),
            pltpu.VMEM((2, 16), jnp.int32),
            pltpu.SMEM((2,), jnp.int32),
            pltpu.SemaphoreType.DMA,
            pltpu.SemaphoreType.DMA,
            pltpu.SemaphoreType.DMA,
        ],
    )
    def p1(labels_hbm, tT_hbm, rows_hbm, slab_v, wlab_v, wpos_v, spos_v,
           blocks_v, offs_v, cur_v, dblk_v, dbase_v, rowbuf_v, idxbuf_v,
           nrow_s, sem, fsem0, fsem1):
        wid = lax.axis_index("s") * NC + lax.axis_index("c")
        lo = wid * bpt
        nblk = jnp.minimum(lo + bpt, blocks) - lo
        pltpu.sync_copy(labels_hbm, slab_v.at[pl.ds(0, B)])
        lane = lax.iota(jnp.int32, L)
        dump = jnp.full((L,), B, jnp.int32)
        idxbuf_v[0, :] = dump
        idxbuf_v[1, :] = dump
        nrow_s[0] = 0
        nrow_s[1] = 0
        zeros = jnp.zeros((L,), jnp.int32)
        for kz in range(272 // L):
            offs_v[pl.ds(kz * L, L)] = zeros
            cur_v[pl.ds(kz * L, L)] = zeros

        def at1(ref, j):
            return ref[pl.ds(j, L)][0]

        def put1(ref, idx, val):
            plsc.store_scatter(ref, [jnp.full((L,), idx, jnp.int32)],
                               jnp.full((L,), val, jnp.int32),
                               mask=lane == 0)

        def scan_body(p, cur):
            v = slab_v[pl.ds(p * L, L)]
            m = ((v >> 7) >= lo) & ((v >> 7) < lo + nblk)
            c = plsc.cumsum(m.astype(jnp.int32))
            at = jnp.full((L,), cur, jnp.int32) + c - 1
            plsc.store_scatter(wlab_v, [at], v, mask=m)
            plsc.store_scatter(wpos_v, [at], lane + p * L, mask=m)
            return cur + c[15]

        n = lax.fori_loop(0, B // L, scan_body, 0)

        ones = jnp.full((L,), 1, jnp.int32)

        def count_body(q, _):
            wv = wlab_v[pl.ds(q * L, L)]
            valid = (lane + q * L) < n
            blk = (wv >> 7) - lo
            blk = jnp.where(valid, blk, 271)
            plsc.addupdate_scatter(cur_v, [blk], ones, mask=valid)
            return 0

        lax.fori_loop(0, (n + L - 1) // L, count_body, 0)

        def psum_body(kz, carry):
            tot, nd = carry
            v = cur_v[pl.ds(kz * L, L)]
            c = plsc.cumsum(v)
            excl = c - v + jnp.full((L,), tot, jnp.int32)
            offs_v[pl.ds(kz * L, L)] = excl
            m = v > 0
            dc = plsc.cumsum(m.astype(jnp.int32))
            dat = jnp.full((L,), nd, jnp.int32) + dc - 1
            plsc.store_scatter(dblk_v, [dat], lane + kz * L, mask=m)
            plsc.store_scatter(dbase_v, [dat], excl, mask=m)
            return tot + c[15], nd + dc[15]

        _, ndist = lax.fori_loop(0, 272 // L, psum_body, (0, 0))
        put1(dbase_v, ndist, n)
        for kz in range(272 // L):
            cur_v[pl.ds(kz * L, L)] = zeros

        def place_body(j, _):
            lab = at1(wlab_v, j)
            blk = (lab >> 7) - lo
            dst = at1(offs_v, blk) + at1(cur_v, blk)
            put1(slab_v, dst, lab)
            put1(spos_v, dst, at1(wpos_v, j))
            put1(cur_v, blk, at1(cur_v, blk) + 1)
            return 0

        lax.fori_loop(0, n, place_body, 0)

        def bdesc(f, u):
            off = pl.multiple_of(
                (at1(dblk_v, f) + lo) * 128, 128)
            return pltpu.make_async_copy(
                tT_hbm.at[:, pl.ds(off, 128)], blocks_v.at[u], sem)

        fsems = (fsem0, fsem1)

        def fdesc(pr, fs):
            return pltpu.make_async_copy(
                rowbuf_v.at[pr], rows_hbm.at[idxbuf_v.at[pr]], fs)

        def flush():
            fc = nrow_s[1]
            for pr in range(2):
                @pl.when((fc & 1) == pr)
                def _(pr=pr):
                    @pl.when(fc >= 2)
                    def _():
                        fdesc(pr, fsems[pr]).wait()
                    fdesc(pr, fsems[pr]).start()
            nrow_s[1] = fc + 1

        def fill_pair():
            return nrow_s[1] & 1

        def process(u):
            def eat(j, _):
                lab = at1(slab_v, j)
                b = at1(spos_v, j)
                lsp = jnp.full((L,), lab & 127, jnp.int32)
                nr = nrow_s[0]
                nrsp = jnp.full((L,), nr, jnp.int32)
                prsp = jnp.full((L,), fill_pair(), jnp.int32)
                for cg in range(D // L):
                    cvec = lane + cg * L
                    val = plsc.load_gather(blocks_v.at[u], [cvec, lsp])
                    plsc.store_scatter(rowbuf_v, [prsp, nrsp, cvec], val)
                plsc.store_scatter(idxbuf_v, [prsp, nrsp],
                                   jnp.full((L,), b, jnp.int32),
                                   mask=lane == 0)
                nrow_s[0] = nr + 1

                @pl.when(nr + 1 == 16)
                def _():
                    flush()
                    nrow_s[0] = 0
                return 0
            return eat

        for u in range(NB):
            @pl.when(u < ndist)
            def _(u=u):
                bdesc(u, u).start()

        def round_body(r, _):
            for u in range(NB):
                f = r * NB + u

                @pl.when(f < ndist)
                def _(f=f, u=u):
                    bdesc(f, u).wait()
                    lax.fori_loop(at1(dbase_v, f), at1(dbase_v, f + 1),
                                  process(u), 0)

                    @pl.when(f + NB < ndist)
                    def _():
                        bdesc(f + NB, u).start()
            return 0

        lax.fori_loop(0, rounds, round_body, 0)
        fc = nrow_s[1]

        @pl.when(nrow_s[0] > 0)
        def _():
            pr = fc & 1
            for prs in range(2):
                @pl.when(pr == prs)
                def _(prs=prs):
                    @pl.when(fc >= 2)
                    def _():
                        fdesc(prs, fsems[prs]).wait()
                    plsc.store_scatter(
                        idxbuf_v, [jnp.full((L,), prs, jnp.int32), lane],
                        dump, mask=lane >= nrow_s[0])
                    pltpu.sync_copy(rowbuf_v.at[prs],
                                    rows_hbm.at[idxbuf_v.at[prs]])

        @pl.when(fc >= 1)
        def _():
            for prs in range(2):
                @pl.when(((fc - 1) & 1) == prs)
                def _(prs=prs):
                    fdesc(prs, fsems[prs]).wait()

        @pl.when(fc >= 2)
        def _():
            for prs in range(2):
                @pl.when((fc & 1) == prs)
                def _(prs=prs):
                    @pl.when(nrow_s[0] == 0)
                    def _():
                        fdesc(prs, fsems[prs]).wait()

    @functools.partial(
        pl.kernel,
        mesh=mesh,
        out_type=jax.ShapeDtypeStruct((D, B), jnp.float32),
        compiler_params=params,
        scratch_types=[
            pltpu.VMEM((b_per_w, 128), jnp.float32),
            pltpu.VMEM((b_per_w // 128, D, 128), jnp.float32),
            pltpu.SemaphoreType.DMA,
        ],
    )
    def p2(rows_hbm, outT_hbm, rbuf_v, obuf_v, sem):
        wid = lax.axis_index("s") * NC + lax.axis_index("c")
        base = wid * b_per_w
        loads = [
            pltpu.async_copy(
                rows_hbm.at[pl.ds(base + g * 128, 128)],
                rbuf_v.at[pl.ds(g * 128, 128)], sem)
            for g in range(b_per_w // 128)
        ]
        lane = lax.iota(jnp.int32, L)
        owrites = []
        for g in range(b_per_w // 128):
            loads[g].wait()
            gsp = jnp.full((L,), g, jnp.int32)

            def cbody(cc, _, gsp=gsp, g=g):
                csp = jnp.full((L,), cc, jnp.int32)
                for jg in range(128 // L):
                    jvec = lane + (g * 128 + jg * L)
                    val = plsc.load_gather(rbuf_v, [jvec, csp])
                    plsc.store_scatter(
                        obuf_v, [gsp, csp, lane + jg * L], val)
                return 0

            lax.fori_loop(0, D, cbody, 0)
            owrites.append(pltpu.async_copy(
                obuf_v.at[g],
                outT_hbm.at[:, pl.ds(base + g * 128, 128)], sem))
        for o in owrites:
            o.wait()

    return p1, p2


def kernel(labels, train, table):
    p1, p2 = _make_kernels(BATCH, HIDDEN, table.shape[0])
    rows = p1(labels.astype(jnp.int32), table.T)
    outT = p2(rows)
    return outT.T
